# baseline (speedup 1.0000x reference)
import functools

import jax
import jax.numpy as jnp
from jax import lax
from jax.experimental import pallas as pl
from jax.experimental.pallas import tpu as pltpu
from jax.experimental.pallas import tpu_sc as plsc

B = 16384
EMB = 64
NUM_CORES = 2
NUM_SUBCORES = 16
NW = NUM_CORES * NUM_SUBCORES
BPW = B // NW


def _tc_pack_tables(Ug, Um, Ig, Im):
    n = Ug.shape[0]
    blk = 5000

    def body(ug_ref, um_ref, ig_ref, im_ref, p1_ref, p2_ref):
        p1_ref[...] = jnp.concatenate([ug_ref[...], um_ref[...]], axis=1)
        p2_ref[...] = jnp.concatenate([ig_ref[...], im_ref[...]], axis=1)

    spec64 = pl.BlockSpec((blk, EMB), lambda i: (i, 0))
    spec128 = pl.BlockSpec((blk, 2 * EMB), lambda i: (i, 0))
    return pl.pallas_call(
        body,
        grid=(n // blk,),
        in_specs=[spec64] * 4,
        out_specs=[spec128] * 2,
        out_shape=[jax.ShapeDtypeStruct((n, 2 * EMB), jnp.float32)] * 2,
    )(Ug, Um, Ig, Im)


def _sc_gather(ids, table):
    mesh = plsc.VectorSubcoreMesh(core_axis_name="c", subcore_axis_name="s")

    @functools.partial(
        pl.kernel,
        mesh=mesh,
        out_type=jax.ShapeDtypeStruct((B, 2 * EMB), jnp.float32),
        scratch_types=[
            pltpu.VMEM((BPW,), jnp.int32),
            pltpu.VMEM((BPW, 2 * EMB), jnp.float32),
            pltpu.SemaphoreType.DMA,
        ],
    )
    def k(ids_hbm, tab_hbm, out_hbm, idx, buf, sem):
        wid = lax.axis_index("s") * NUM_CORES + lax.axis_index("c")
        base = wid * BPW
        rows = pl.ds(base, BPW)
        pltpu.sync_copy(ids_hbm.at[rows], idx)
        pltpu.async_copy(tab_hbm.at[idx], buf, sem).wait()
        pltpu.sync_copy(buf, out_hbm.at[rows])

    return k(ids, table)


def _tc_mlp_final(u, i_rows, W1, b1, W2, b2, W3, b3, Wp, bp):
    w1u = W1[:, :EMB].T
    w1i = W1[:, EMB:].T
    w2t = W2.T
    w3t = W3.T
    wpg = Wp[:, :EMB]
    wpx = Wp[:, EMB:]
    b1r = b1.reshape(1, -1)
    b2r = b2.reshape(1, -1)
    b3r = b3.reshape(1, -1)
    bpr = jnp.reshape(bp, (1, 1))

    BLK = 2048
    h0 = W1.shape[0]
    h1 = W2.shape[0]
    h2 = W3.shape[0]

    def body(u_ref, i_ref, w1u_ref, w1i_ref, b1_ref, w2_ref, b2_ref,
             w3_ref, b3_ref, wpg_ref, wpx_ref, bp_ref, out_ref):
        uu = u_ref[...]
        ii = i_ref[...]
        um = uu[:, EMB:]
        im = ii[:, EMB:]
        gmf = uu[:, :EMB] * ii[:, :EMB]
        x = jnp.dot(um, w1u_ref[...], preferred_element_type=jnp.float32)
        x = x + jnp.dot(im, w1i_ref[...], preferred_element_type=jnp.float32)
        x = jnp.maximum(x + b1_ref[...], 0.0)
        x = jnp.dot(x, w2_ref[...], preferred_element_type=jnp.float32)
        x = jnp.maximum(x + b2_ref[...], 0.0)
        x = jnp.dot(x, w3_ref[...], preferred_element_type=jnp.float32)
        x = jnp.maximum(x + b3_ref[...], 0.0)
        logit = (jnp.sum(gmf * wpg_ref[...], axis=1, keepdims=True)
                 + jnp.sum(x * wpx_ref[...], axis=1, keepdims=True)
                 + bp_ref[0, 0])
        out_ref[...] = 1.0 / (1.0 + jnp.exp(-logit))

    full = lambda r, c: pl.BlockSpec((r, c), lambda i: (0, 0))
    out = pl.pallas_call(
        body,
        grid=(B // BLK,),
        in_specs=[
            pl.BlockSpec((BLK, 2 * EMB), lambda i: (i, 0)),
            pl.BlockSpec((BLK, 2 * EMB), lambda i: (i, 0)),
            full(EMB, h0),
            full(EMB, h0),
            full(1, h0),
            full(h0, h1),
            full(1, h1),
            full(h1, h2),
            full(1, h2),
            full(1, EMB),
            full(1, h2),
            full(1, 1),
        ],
        out_specs=pl.BlockSpec((BLK, 1), lambda i: (i, 0)),
        out_shape=jax.ShapeDtypeStruct((B, 1), jnp.float32),
    )(u, i_rows, w1u, w1i, b1r, w2t, b2r, w3t, b3r, wpg, wpx, bpr)
    return jnp.squeeze(out, axis=-1)


def kernel(user_ids, item_ids, Ug, Ig, Um, Im, W1, b1, W2, b2, W3, b3, Wp, bp):
    uid = user_ids.astype(jnp.int32)
    iid = item_ids.astype(jnp.int32)
    p1, p2 = _tc_pack_tables(Ug, Um, Ig, Im)
    u = _sc_gather(uid, p1)
    i_rows = _sc_gather(iid, p2)
    return _tc_mlp_final(u, i_rows, W1, b1, W2, b2, W3, b3, Wp, bp)


# speedup vs baseline: 1.2766x; 1.2766x over previous
import functools

import jax
import jax.numpy as jnp
from jax import lax
from jax.experimental import pallas as pl
from jax.experimental.pallas import tpu as pltpu
from jax.experimental.pallas import tpu_sc as plsc

B = 16384
EMB = 64
NUM_CORES = 2
NUM_SUBCORES = 16
NW = NUM_CORES * NUM_SUBCORES
BPW = B // NW


def _sc_gather(ids, table):
    mesh = plsc.VectorSubcoreMesh(core_axis_name="c", subcore_axis_name="s")

    @functools.partial(
        pl.kernel,
        mesh=mesh,
        out_type=jax.ShapeDtypeStruct((B, 2 * EMB), jnp.float32),
        scratch_types=[
            pltpu.VMEM((BPW,), jnp.int32),
            pltpu.VMEM((BPW, 2 * EMB), jnp.float32),
            pltpu.SemaphoreType.DMA,
        ],
    )
    def k(ids_hbm, tab_hbm, out_hbm, idx, buf, sem):
        wid = lax.axis_index("s") * NUM_CORES + lax.axis_index("c")
        base = wid * BPW
        rows = pl.ds(base, BPW)
        pltpu.sync_copy(ids_hbm.at[rows], idx)
        pltpu.async_copy(tab_hbm.at[idx], buf, sem).wait()
        pltpu.sync_copy(buf, out_hbm.at[rows])

    return k(ids, table)


def _tc_mlp_final(u, i_rows, W1, b1, W2, b2, W3, b3, Wp, bp):
    w1u = W1[:, :EMB].T
    w1i = W1[:, EMB:].T
    w2t = W2.T
    w3t = W3.T
    wpg = Wp[:, :EMB]
    wpx = Wp[:, EMB:]
    b1r = b1.reshape(1, -1)
    b2r = b2.reshape(1, -1)
    b3r = b3.reshape(1, -1)
    bpr = jnp.reshape(bp, (1, 1))

    BLK = 2048
    h0 = W1.shape[0]
    h1 = W2.shape[0]
    h2 = W3.shape[0]

    def body(u_ref, i_ref, w1u_ref, w1i_ref, b1_ref, w2_ref, b2_ref,
             w3_ref, b3_ref, wpg_ref, wpx_ref, bp_ref, out_ref):
        uu = u_ref[...]
        ii = i_ref[...]
        um = uu[:, EMB:]
        im = ii[:, EMB:]
        gmf = uu[:, :EMB] * ii[:, :EMB]
        x = jnp.dot(um, w1u_ref[...], preferred_element_type=jnp.float32)
        x = x + jnp.dot(im, w1i_ref[...], preferred_element_type=jnp.float32)
        x = jnp.maximum(x + b1_ref[...], 0.0)
        x = jnp.dot(x, w2_ref[...], preferred_element_type=jnp.float32)
        x = jnp.maximum(x + b2_ref[...], 0.0)
        x = jnp.dot(x, w3_ref[...], preferred_element_type=jnp.float32)
        x = jnp.maximum(x + b3_ref[...], 0.0)
        logit = (jnp.sum(gmf * wpg_ref[...], axis=1, keepdims=True)
                 + jnp.sum(x * wpx_ref[...], axis=1, keepdims=True)
                 + bp_ref[0, 0])
        out_ref[...] = 1.0 / (1.0 + jnp.exp(-logit))

    full = lambda r, c: pl.BlockSpec((r, c), lambda i: (0, 0))
    out = pl.pallas_call(
        body,
        grid=(B // BLK,),
        in_specs=[
            pl.BlockSpec((BLK, 2 * EMB), lambda i: (i, 0)),
            pl.BlockSpec((BLK, 2 * EMB), lambda i: (i, 0)),
            full(EMB, h0),
            full(EMB, h0),
            full(1, h0),
            full(h0, h1),
            full(1, h1),
            full(h1, h2),
            full(1, h2),
            full(1, EMB),
            full(1, h2),
            full(1, 1),
        ],
        out_specs=pl.BlockSpec((BLK, 1), lambda i: (i, 0)),
        out_shape=jax.ShapeDtypeStruct((B, 1), jnp.float32),
    )(u, i_rows, w1u, w1i, b1r, w2t, b2r, w3t, b3r, wpg, wpx, bpr)
    return jnp.squeeze(out, axis=-1)


def kernel(user_ids, item_ids, Ug, Ig, Um, Im, W1, b1, W2, b2, W3, b3, Wp, bp):
    uid = user_ids.astype(jnp.int32)
    iid = item_ids.astype(jnp.int32)
    p1 = jnp.concatenate([Ug, Um], axis=1)
    u = _sc_gather(uid, p1)
    p2 = jnp.concatenate([Ig, Im], axis=1)
    i_rows = _sc_gather(iid, p2)
    return _tc_mlp_final(u, i_rows, W1, b1, W2, b2, W3, b3, Wp, bp)
